# baseline (speedup 1.0000x reference)
import functools

import jax
import jax.numpy as jnp
from jax import lax
from jax.experimental import pallas as pl
from jax.experimental.pallas import tpu as pltpu
from jax.experimental.pallas import tpu_sc as plsc

_V = 1000000
_D = 64
_R = 8
_LANES = 16
_NDB = _D // _LANES
_NW = 32
_BPW = 128
_H = 50
_PKC = 512


def _pack_body(e_ref, a_ref, o_ref):
    eye_d = jnp.eye(_D, dtype=jnp.float32)
    eye_r = jnp.eye(_R, dtype=jnp.float32)
    et = lax.dot_general(e_ref[...], eye_d, (((0,), (0,)), ((), ())),
                         precision=lax.Precision.HIGHEST)
    at = lax.dot_general(a_ref[...], eye_r, (((0,), (0,)), ((), ())),
                         precision=lax.Precision.HIGHEST)
    o_ref[:, : _D] = et
    for i in range(_D // _R):
        o_ref[:, pl.ds(_D + i * _R, _R)] = at


def _pack_tables(emb_t, a_t):
    grid = (_V + _PKC - 1) // _PKC
    return pl.pallas_call(
        _pack_body,
        grid=(grid,),
        in_specs=[
            pl.BlockSpec((_D, _PKC), lambda g: (0, g)),
            pl.BlockSpec((_R, _PKC), lambda g: (0, g)),
        ],
        out_specs=pl.BlockSpec((_PKC, 2 * _D), lambda g: (g, 0)),
        out_shape=jax.ShapeDtypeStruct((_V, 2 * _D), jnp.float32),
    )(emb_t, a_t)


def _make_sc_kernel():
    mesh = plsc.VectorSubcoreMesh(core_axis_name="c", subcore_axis_name="s")

    @functools.partial(
        pl.kernel,
        mesh=mesh,
        compiler_params=pltpu.CompilerParams(needs_layout_passes=False),
        out_type=jax.ShapeDtypeStruct((_H, _D // 8, _NW, 8, _BPW),
                                      jnp.float32),
        scratch_types=[
            pltpu.VMEM((_H, _BPW), jnp.int32),
            pltpu.VMEM((2, _BPW, 2 * _D), jnp.float32),
            pltpu.VMEM((2, _D // 8, 8, _BPW + 1), jnp.float32),
            pltpu.VMEM((_R, 2 * _D), jnp.float32),
            pltpu.SemaphoreType.DMA,
            pltpu.SemaphoreType.DMA,
            pltpu.SemaphoreType.DMA,
            pltpu.SemaphoreType.DMA,
        ],
    )
    def sc_kernel(idx_hbm, tab_hbm, b_hbm, out_hbm,
                  idx_v, rows2_v, tr2_v, b_v, gs0, gs1, os0, os1):
        num_cores = 2
        wid = lax.axis_index("s") * num_cores + lax.axis_index("c")

        pltpu.sync_copy(idx_hbm.at[:, wid], idx_v)
        pltpu.sync_copy(b_hbm, b_v)

        b_vecs = [[b_v[r, pl.ds(db * _LANES, _LANES)] for db in range(_NDB)]
                  for r in range(_R)]
        j_vecs = [lax.iota(jnp.int32, _LANES) + db * _LANES
                  for db in range(_NDB)]
        jt_vecs = [jv // 8 for jv in j_vecs]
        j8_vecs = [jv % 8 for jv in j_vecs]

        bufs = [(rows2_v.at[0], tr2_v.at[0], gs0, os0),
                (rows2_v.at[1], tr2_v.at[1], gs1, os1)]

        def gather(h, buf):
            rows_v, _, gsem, _ = bufs[buf]
            return pltpu.make_async_copy(tab_hbm.at[idx_v.at[h]],
                                         rows_v, gsem)

        def out_copy(h, buf):
            _, tr_v, _, osem = bufs[buf]
            return pltpu.make_async_copy(tr_v.at[:, :, pl.ds(0, _BPW)],
                                         out_hbm.at[h, :, wid], osem)

        def compute_unit(h, buf):
            rows_v, tr_v, _, _ = bufs[buf]

            def row_body(k, c):
                k_vec = jnp.full((_LANES,), k, jnp.int32)
                avs = [plsc.load_gather(
                           rows_v,
                           [k_vec, jnp.full((_LANES,), _D + r, jnp.int32)])
                       for r in range(_R)]
                for db in range(_NDB):
                    acc = rows_v[k, pl.ds(db * _LANES, _LANES)]
                    for r in range(_R):
                        acc = acc + avs[r] * b_vecs[r][db]
                    plsc.store_scatter(
                        tr_v, [jt_vecs[db], j8_vecs[db], k_vec], acc)
                return c

            lax.fori_loop(0, _BPW, row_body, 0)

        gather(0, 0).start()

        def pair_body(hh, carry):
            for b in (0, 1):
                h = hh * 2 + b

                @pl.when(h + 1 < _H)
                def _():
                    gather(h + 1, 1 - b).start()

                gather(h, b).wait()

                @pl.when(h >= 2)
                def _():
                    out_copy(h - 2, b).wait()

                compute_unit(h, b)
                out_copy(h, b).start()
            return carry

        lax.fori_loop(0, _H // 2, pair_body, 0)
        out_copy(_H - 2, 0).wait()
        out_copy(_H - 1, 1).wait()

    return sc_kernel


_sc_kernel = _make_sc_kernel()


def kernel(inputs, embeddings, lora_A, lora_B):
    batch, hist = inputs.shape
    packed = _pack_tables(embeddings.T, lora_A.T)
    idx3 = inputs.T.reshape(_H, _NW, _BPW)
    b_scaled = lora_B * jnp.sqrt(jnp.asarray(_D, jnp.float32))
    b_pad = jnp.concatenate([b_scaled, b_scaled], axis=1)
    out5 = _sc_kernel(idx3, packed, b_pad)
    out = out5.transpose(2, 4, 0, 1, 3).reshape(batch, hist, _D)
    return out


# speedup vs baseline: 2.3210x; 2.3210x over previous
import functools

import jax
import jax.numpy as jnp
from jax import lax
from jax.experimental import pallas as pl
from jax.experimental.pallas import tpu as pltpu
from jax.experimental.pallas import tpu_sc as plsc

_V = 1000000
_D = 64
_R = 8
_LANES = 16
_NDB = _D // _LANES
_NW = 32
_BPW = 128
_H = 50


def _make_sc_kernel():
    mesh = plsc.VectorSubcoreMesh(core_axis_name="c", subcore_axis_name="s")

    @functools.partial(
        pl.kernel,
        mesh=mesh,
        compiler_params=pltpu.CompilerParams(use_tc_tiling_on_sc=False,
                                             needs_layout_passes=False),
        out_type=jax.ShapeDtypeStruct((_H, _D // 8, _NW, 8, _BPW),
                                      jnp.float32),
        scratch_types=[
            pltpu.VMEM((_H, _BPW), jnp.int32),
            pltpu.VMEM((2, _BPW, _D), jnp.float32),
            pltpu.VMEM((2, _BPW, _R), jnp.float32),
            pltpu.VMEM((2, _D // 8, 8, _BPW + 1), jnp.float32),
            pltpu.VMEM((_R, _D), jnp.float32),
            pltpu.SemaphoreType.DMA,
            pltpu.SemaphoreType.DMA,
            pltpu.SemaphoreType.DMA,
            pltpu.SemaphoreType.DMA,
        ],
    )
    def sc_kernel(idx_hbm, emb_hbm, a_hbm, b_hbm, out_hbm,
                  idx_v, rows2_v, av2_v, tr2_v, b_v, gs0, gs1, os0, os1):
        num_cores = 2
        wid = lax.axis_index("s") * num_cores + lax.axis_index("c")

        pltpu.sync_copy(idx_hbm.at[:, pl.ds(wid * _BPW, _BPW)], idx_v)
        pltpu.sync_copy(b_hbm, b_v)

        b_vecs = [[b_v[r, pl.ds(db * _LANES, _LANES)] for db in range(_NDB)]
                  for r in range(_R)]
        j_vecs = [lax.iota(jnp.int32, _LANES) + db * _LANES
                  for db in range(_NDB)]
        jt_vecs = [jv // 8 for jv in j_vecs]
        j8_vecs = [jv % 8 for jv in j_vecs]

        bufs = [(rows2_v.at[0], av2_v.at[0], tr2_v.at[0], gs0, os0),
                (rows2_v.at[1], av2_v.at[1], tr2_v.at[1], gs1, os1)]

        def gathers(h, buf):
            rows_v, av_v, _, gsem, _ = bufs[buf]
            return [pltpu.make_async_copy(emb_hbm.at[idx_v.at[h]],
                                          rows_v, gsem),
                    pltpu.make_async_copy(a_hbm.at[idx_v.at[h]],
                                          av_v, gsem)]

        def out_copy(h, buf):
            _, _, tr_v, _, osem = bufs[buf]
            return pltpu.make_async_copy(tr_v.at[:, :, pl.ds(0, _BPW)],
                                         out_hbm.at[h, :, wid], osem)

        def compute_unit(h, buf):
            rows_v, av_v, tr_v, _, _ = bufs[buf]

            def row_body(k, c):
                k_vec = jnp.full((_LANES,), k, jnp.int32)
                avs = [plsc.load_gather(
                           av_v, [k_vec, jnp.full((_LANES,), r, jnp.int32)])
                       for r in range(_R)]
                for db in range(_NDB):
                    acc = rows_v[k, pl.ds(db * _LANES, _LANES)]
                    for r in range(_R):
                        acc = acc + avs[r] * b_vecs[r][db]
                    plsc.store_scatter(
                        tr_v, [jt_vecs[db], j8_vecs[db], k_vec], acc)
                return c

            lax.fori_loop(0, _BPW, row_body, 0)

        for cp in gathers(0, 0):
            cp.start()

        def pair_body(hh, carry):
            for b in (0, 1):
                h = hh * 2 + b

                @pl.when(h + 1 < _H)
                def _():
                    for cp in gathers(h + 1, 1 - b):
                        cp.start()

                for cp in gathers(h, b):
                    cp.wait()

                @pl.when(h >= 2)
                def _():
                    out_copy(h - 2, b).wait()

                compute_unit(h, b)
                out_copy(h, b).start()
            return carry

        lax.fori_loop(0, _H // 2, pair_body, 0)
        out_copy(_H - 2, 0).wait()
        out_copy(_H - 1, 1).wait()

    return sc_kernel


_sc_kernel = _make_sc_kernel()


def kernel(inputs, embeddings, lora_A, lora_B):
    batch, hist = inputs.shape
    idx_t = inputs.T
    b_scaled = lora_B * jnp.sqrt(jnp.asarray(_D, jnp.float32))
    out5 = _sc_kernel(idx_t, embeddings, lora_A, b_scaled)
    out = out5.transpose(2, 4, 0, 1, 3).reshape(batch, hist, _D)
    return out
